# baseline (speedup 1.0000x reference)
import functools
import math

import numpy as np
import jax
import jax.numpy as jnp
from jax import lax
from jax.experimental import pallas as pl
from jax.experimental.pallas import tpu as pltpu
from jax.experimental.pallas import tpu_sc as plsc

D_MODEL = 2048
SEQ_LEN = 4096
BATCH = 2

_NC = 2
_NS = 16
_LANES = 16
_NW = _NC * _NS
_LPW = SEQ_LEN // _NW
_CL = 16
_VPR = D_MODEL // _LANES


def _pe_const():
    position = np.arange(0, SEQ_LEN, dtype=np.float32)[:, None]
    div_term = np.exp(
        np.arange(0, D_MODEL, 2, dtype=np.float32) * -(math.log(10000.0) / D_MODEL)
    )
    pe = np.zeros((SEQ_LEN, D_MODEL), dtype=np.float32)
    pe[:, 0::2] = np.sin(position * div_term)
    pe[:, 1::2] = np.cos(position * div_term)
    return jnp.asarray(pe)


def _body(x_hbm, table_hbm, pe_hbm, out_hbm,
          idx0_v, idx1_v, pe_v, rows0_v, rows1_v, sem0, sem1, semp):
    wid = lax.axis_index("s") * _NC + lax.axis_index("c")
    lbase = wid * _LPW

    def chunk(c, carry):
        off = lbase + c * _CL

        pltpu.sync_copy(x_hbm.at[pl.ds(off, _CL)], idx0_v)
        pltpu.sync_copy(x_hbm.at[pl.ds(SEQ_LEN + off, _CL)], idx1_v)

        cpe = pltpu.async_copy(pe_hbm.at[pl.ds(off, _CL)], pe_v, semp)
        c0 = pltpu.async_copy(table_hbm.at[idx0_v], rows0_v, sem0)
        c1 = pltpu.async_copy(table_hbm.at[idx1_v], rows1_v, sem1)
        cpe.wait()
        c0.wait()
        c1.wait()

        def add_row(r, _):
            def add_vec(j, _):
                s = j * _LANES
                p = pe_v[r, pl.ds(s, _LANES)]
                rows0_v[r, pl.ds(s, _LANES)] = rows0_v[r, pl.ds(s, _LANES)] + p
                rows1_v[r, pl.ds(s, _LANES)] = rows1_v[r, pl.ds(s, _LANES)] + p
                return 0
            return lax.fori_loop(0, _VPR, add_vec, 0)

        lax.fori_loop(0, _CL, add_row, 0)

        pltpu.sync_copy(rows0_v, out_hbm.at[pl.ds(off, _CL)])
        pltpu.sync_copy(rows1_v, out_hbm.at[pl.ds(SEQ_LEN + off, _CL)])
        return carry

    lax.fori_loop(0, _LPW // _CL, chunk, 0)


@functools.partial(jax.jit, static_argnames=())
def _run(xf, table, pe):
    mesh = plsc.VectorSubcoreMesh(core_axis_name="c", subcore_axis_name="s")
    f = pl.kernel(
        _body,
        out_type=jax.ShapeDtypeStruct((BATCH * SEQ_LEN, D_MODEL), jnp.float32),
        mesh=mesh,
        scratch_types=[
            pltpu.VMEM((_CL,), jnp.int32),
            pltpu.VMEM((_CL,), jnp.int32),
            pltpu.VMEM((_CL, D_MODEL), jnp.float32),
            pltpu.VMEM((_CL, D_MODEL), jnp.float32),
            pltpu.VMEM((_CL, D_MODEL), jnp.float32),
            pltpu.SemaphoreType.DMA,
            pltpu.SemaphoreType.DMA,
            pltpu.SemaphoreType.DMA,
        ],
    )
    return f(xf, table, pe)


def kernel(x, table):
    xf = x.reshape(BATCH * SEQ_LEN).astype(jnp.int32)
    pe = _pe_const()
    out = _run(xf, table, pe)
    return out.reshape(BATCH, SEQ_LEN, D_MODEL)


# speedup vs baseline: 1.1891x; 1.1891x over previous
import functools
import math

import numpy as np
import jax
import jax.numpy as jnp
from jax import lax
from jax.experimental import pallas as pl
from jax.experimental.pallas import tpu as pltpu
from jax.experimental.pallas import tpu_sc as plsc

D_MODEL = 2048
SEQ_LEN = 4096
BATCH = 2

_NC = 2
_NS = 16
_LANES = 16
_NW = _NC * _NS
_LPW = SEQ_LEN // _NW
_CL = 8
_NCH = _LPW // _CL
_VPR = D_MODEL // _LANES


def _pe_const():
    position = np.arange(0, SEQ_LEN, dtype=np.float32)[:, None]
    div_term = np.exp(
        np.arange(0, D_MODEL, 2, dtype=np.float32) * -(math.log(10000.0) / D_MODEL)
    )
    pe = np.zeros((SEQ_LEN, D_MODEL), dtype=np.float32)
    pe[:, 0::2] = np.sin(position * div_term)
    pe[:, 1::2] = np.cos(position * div_term)
    return jnp.asarray(pe)


def _body(x_hbm, table_hbm, pe_hbm, out_hbm,
          idx0_v, idx1_v, pe0_v, pe1_v, rows0_v, rows1_v,
          g0_sem, g1_sem, p0_sem, p1_sem, s0_sem, s1_sem):
    wid = lax.axis_index("s") * _NC + lax.axis_index("c")
    lbase = wid * _LPW

    idx = (idx0_v, idx1_v)
    pe = (pe0_v, pe1_v)
    rows = (rows0_v, rows1_v)
    g_sem = (g0_sem, g1_sem)
    p_sem = (p0_sem, p1_sem)
    s_sem = (s0_sem, s1_sem)

    def issue_load(c, s):
        off = lbase + c * _CL
        pltpu.sync_copy(x_hbm.at[pl.ds(off, _CL)], idx[s].at[0])
        pltpu.sync_copy(x_hbm.at[pl.ds(SEQ_LEN + off, _CL)], idx[s].at[1])
        pltpu.async_copy(pe_hbm.at[pl.ds(off, _CL)], pe[s], p_sem[s])
        pltpu.async_copy(table_hbm.at[idx[s].at[0]], rows[s].at[0], g_sem[s])
        pltpu.async_copy(table_hbm.at[idx[s].at[1]], rows[s].at[1], g_sem[s])

    def wait_load(s):
        pltpu.make_async_copy(pe_hbm.at[pl.ds(0, _CL)], pe[s], p_sem[s]).wait()
        pltpu.make_async_copy(table_hbm.at[idx[s].at[0]], rows[s].at[0], g_sem[s]).wait()
        pltpu.make_async_copy(table_hbm.at[idx[s].at[1]], rows[s].at[1], g_sem[s]).wait()

    def issue_store(c, s):
        off = lbase + c * _CL
        pltpu.async_copy(rows[s].at[0], out_hbm.at[pl.ds(off, _CL)], s_sem[s])
        pltpu.async_copy(rows[s].at[1], out_hbm.at[pl.ds(SEQ_LEN + off, _CL)], s_sem[s])

    def wait_store(s):
        pltpu.make_async_copy(rows[s].at[0], out_hbm.at[pl.ds(0, _CL)], s_sem[s]).wait()
        pltpu.make_async_copy(rows[s].at[1], out_hbm.at[pl.ds(0, _CL)], s_sem[s]).wait()

    def compute(s):
        r0 = rows[s].at[0]
        r1 = rows[s].at[1]
        pv = pe[s]

        def add_row(r, _):
            def add_vec(j, _):
                d = pl.ds(j * _LANES, _LANES)
                p = pv[r, d]
                r0[r, d] = r0[r, d] + p
                r1[r, d] = r1[r, d] + p
                return 0
            return lax.fori_loop(0, _VPR, add_vec, 0)

        lax.fori_loop(0, _CL, add_row, 0)

    issue_load(0, 0)
    for c in range(_NCH):
        s = c % 2
        if c + 1 < _NCH:
            if c >= 1:
                wait_store(1 - s)
            issue_load(c + 1, 1 - s)
        wait_load(s)
        compute(s)
        issue_store(c, s)
    wait_store(_NCH % 2)
    wait_store(1 - (_NCH % 2))


@jax.jit
def _run(xf, table, pe):
    mesh = plsc.VectorSubcoreMesh(core_axis_name="c", subcore_axis_name="s")
    f = pl.kernel(
        _body,
        out_type=jax.ShapeDtypeStruct((BATCH * SEQ_LEN, D_MODEL), jnp.float32),
        mesh=mesh,
        scratch_types=[
            pltpu.VMEM((2, _CL), jnp.int32),
            pltpu.VMEM((2, _CL), jnp.int32),
            pltpu.VMEM((_CL, D_MODEL), jnp.float32),
            pltpu.VMEM((_CL, D_MODEL), jnp.float32),
            pltpu.VMEM((2, _CL, D_MODEL), jnp.float32),
            pltpu.VMEM((2, _CL, D_MODEL), jnp.float32),
            pltpu.SemaphoreType.DMA,
            pltpu.SemaphoreType.DMA,
            pltpu.SemaphoreType.DMA,
            pltpu.SemaphoreType.DMA,
            pltpu.SemaphoreType.DMA,
            pltpu.SemaphoreType.DMA,
        ],
    )
    return f(xf, table, pe)


def kernel(x, table):
    xf = x.reshape(BATCH * SEQ_LEN).astype(jnp.int32)
    pe = _pe_const()
    out = _run(xf, table, pe)
    return out.reshape(BATCH, SEQ_LEN, D_MODEL)
